# baseline (speedup 1.0000x reference)
import functools

import jax
import jax.numpy as jnp
from jax import lax
from jax.experimental import pallas as pl
from jax.experimental.pallas import tpu as pltpu
from jax.experimental.pallas import tpu_sc as plsc

B = 4096
L = 2048
K = 0.2
NW = 32
RPW = B // NW
RPC = 8
NCH = RPW // RPC
NV = L // 16
UNROLL = 8
NT_CONF = 5.0
NEG_BIG = -12111.0


_GATHER_DNUMS = lax.GatherDimensionNumbers(
    offset_dims=(), collapsed_slice_dims=(0,), start_index_map=(0,))


def _take16(x, idx):
    return lax.gather(x, idx[:, None], _GATHER_DNUMS, slice_sizes=(1,),
                      mode=lax.GatherScatterMode.PROMISE_IN_BOUNDS)


def _lanesum(x):
    lanes = lax.iota(jnp.int32, 16)
    for k in (8, 4, 2, 1):
        x = x + _take16(x, lanes ^ k)
    return x


def _row_loss(mbuf, pmbuf, slot, r):
    mrow = mbuf.at[slot].at[r]
    pmrow = pmbuf.at[slot].at[r]
    zf = jnp.zeros((16,), jnp.float32)
    lanes = lax.iota(jnp.int32, 16)

    shift_idx = jnp.maximum(lanes - 1, 0)
    last_idx = jnp.full((16,), 15, jnp.int32)

    def bodyA(jo, carry):
        sm, sp, tv, prev = carry
        for u in range(UNROLL):
            j = jo * UNROLL + u
            a = mrow[pl.ds(j * 16, 16)]
            p = pmrow[pl.ds(j * 16, 16)]
            s = _take16(a, shift_idx)
            b = jnp.where(lanes == 0, prev, s)
            prev = _take16(a, last_idx)
            sm = sm + a
            sp = sp + p
            tv = tv + jnp.abs(a - b)
        return sm, sp, tv, prev

    first = _take16(mrow[pl.ds(0, 16)], jnp.zeros((16,), jnp.int32))
    sm, sp, tv, _ = lax.fori_loop(0, NV // UNROLL, bodyA, (zf, zf, zf, first))
    m_sum = _lanesum(sm)
    pm_sum = _lanesum(sp)
    tv_sum = _lanesum(tv)
    c = (pm_sum * jnp.float32(K)).astype(jnp.int32)

    def count_ge(cand):
        def cnt_body(jo, acc):
            for u in range(UNROLL):
                j = jo * UNROLL + u
                a = mrow[pl.ds(j * 16, 16)]
                acc = acc + jnp.where(a >= cand, jnp.int32(1), jnp.int32(0))
            return acc

        cntv = lax.fori_loop(0, NV // UNROLL, cnt_body,
                             jnp.zeros((16,), jnp.int32))
        return _lanesum(cntv)

    t = jnp.zeros((16,), jnp.float32)
    for j in range(6, -1, -1):
        mul = jnp.float32(2.0 ** (2 ** j))
        const = jnp.float32(2.0 ** (2 ** j - 127))
        cand = jnp.where(t == 0.0, const, t * mul)
        cnt = count_ge(cand)
        t = jnp.where(cnt >= c, cand, t)
    scale = jnp.maximum(t * jnp.float32(2.0 ** -23), jnp.float32(2.0 ** -149))

    def bodyM(i, carry):
        t, ulp = carry
        cand = t + ulp
        cnt = count_ge(cand)
        t = jnp.where(cnt >= c, cand, t)
        return t, ulp * jnp.float32(0.5)

    t, _ = lax.fori_loop(0, 23, bodyM, (t, scale * jnp.float32(2.0 ** 22)))

    def bodyC(jo, carry):
        cg, sg = carry
        for u in range(UNROLL):
            j = jo * UNROLL + u
            a = mrow[pl.ds(j * 16, 16)]
            gt = a > t
            cg = cg + jnp.where(gt, jnp.int32(1), jnp.int32(0))
            sg = sg + jnp.where(gt, a, jnp.float32(0.0))
        return cg, sg

    cgv, sgv = lax.fori_loop(0, NV // UNROLL, bodyC,
                             (jnp.zeros((16,), jnp.int32), zf))
    cnt_gt = _lanesum(cgv)
    sum_gt = _lanesum(sgv)
    topc = sum_gt + (c - cnt_gt).astype(jnp.float32) * t
    row_spar = jnp.where(c == 0,
                         jnp.float32(L) - m_sum,
                         m_sum + c.astype(jnp.float32) - 2.0 * topc)
    return row_spar + tv_sum / jnp.float32(B)


def _sc_body(m_hbm, pm_hbm, l0_hbm, l1_hbm, tgt_hbm, out_hbm,
             mbuf, pmbuf, l0b, l1b, tgtb, obuf,
             sm0, sm1, sp0, sp1):
    cid = lax.axis_index("c")
    sid = lax.axis_index("s")
    wid = sid * 2 + cid
    base = wid * RPW

    sems = ((sm0, sp0), (sm1, sp1))

    def start(ch, slot):
        r0 = base + ch * RPC
        pltpu.async_copy(m_hbm.at[pl.ds(r0, RPC)], mbuf.at[slot],
                         sems[slot][0])
        pltpu.async_copy(pm_hbm.at[pl.ds(r0, RPC)], pmbuf.at[slot],
                         sems[slot][1])

    def wait(slot):
        pltpu.make_async_copy(m_hbm.at[pl.ds(0, RPC)], mbuf.at[slot],
                              sems[slot][0]).wait()
        pltpu.make_async_copy(pm_hbm.at[pl.ds(0, RPC)], pmbuf.at[slot],
                              sems[slot][1]).wait()

    pltpu.sync_copy(l0_hbm.at[pl.ds(base, RPW)], l0b)
    pltpu.sync_copy(l1_hbm.at[pl.ds(base, RPW)], l1b)
    pltpu.sync_copy(tgt_hbm.at[pl.ds(base, RPW)], tgtb)

    def cw_body(j, acc):
        r0 = l0b[pl.ds(j * 16, 16)]
        r1 = l1b[pl.ds(j * 16, 16)]
        tg = tgtb[pl.ds(j * 16, 16)]
        is0 = tg == 0
        this = jnp.where(is0, r0, r1)
        other = jnp.maximum(jnp.where(is0, r1, r0), jnp.float32(NEG_BIG))
        nt = jnp.maximum(this - other + jnp.float32(NT_CONF), 0.0)
        return acc + nt

    cwv = lax.fori_loop(0, RPW // 16, cw_body, jnp.zeros((16,), jnp.float32))
    part = _lanesum(cwv) / jnp.float32(B)

    start(0, 0)
    start(1, 1)

    def pair_body(g, part):
        ch0 = 2 * g

        def rows(slot, part):
            def row_body(r, p):
                return p + _row_loss(mbuf, pmbuf, slot, r)
            return lax.fori_loop(0, RPC, row_body, part)

        wait(0)
        part = rows(0, part)

        @pl.when(ch0 + 2 < NCH)
        def _():
            start(ch0 + 2, 0)

        wait(1)
        part = rows(1, part)

        @pl.when(ch0 + 3 < NCH)
        def _():
            start(ch0 + 3, 1)

        return part

    part = lax.fori_loop(0, NCH // 2, pair_body, part)

    lanes = lax.iota(jnp.int32, 16)
    obuf[...] = jnp.where(lanes == 0, part, 0.0)
    pltpu.sync_copy(obuf, out_hbm.at[wid])


def sc_loss(masks, position_mask, logits0, logits1, targets):
    mesh = plsc.VectorSubcoreMesh(core_axis_name="c", subcore_axis_name="s")
    f = functools.partial(
        pl.kernel,
        mesh=mesh,
        out_type=jax.ShapeDtypeStruct((NW, 16), jnp.float32),
        scratch_types=[
            pltpu.VMEM((2, RPC, L), jnp.float32),
            pltpu.VMEM((2, RPC, L), jnp.float32),
            pltpu.VMEM((RPW,), jnp.float32),
            pltpu.VMEM((RPW,), jnp.float32),
            pltpu.VMEM((RPW,), jnp.int32),
            pltpu.VMEM((16,), jnp.float32),
            pltpu.SemaphoreType.DMA,
            pltpu.SemaphoreType.DMA,
            pltpu.SemaphoreType.DMA,
            pltpu.SemaphoreType.DMA,
        ],
    )(_sc_body)
    return f(masks, position_mask, logits0, logits1, targets)


def kernel(outputs_support, outputs_delete, targets, position_mask, masks):
    logits = outputs_delete[1]
    l0 = logits[:, 0]
    l1 = logits[:, 1]
    out = sc_loss(masks, position_mask, l0, l1, targets)
    return jnp.sum(out) + outputs_support[0]


# speedup vs baseline: 1.3691x; 1.3691x over previous
import functools

import jax
import jax.numpy as jnp
from jax import lax
from jax.experimental import pallas as pl
from jax.experimental.pallas import tpu as pltpu

B = 4096
L = 2048
K = 0.2
ROW_BLOCK = 512
NUM_BLOCKS = B // ROW_BLOCK
NT_CONF = 5.0
NEG_BIG = -12111.0


def _loss_kernel(logits2_ref, targets_ref, support_ref, pm_ref, m_ref, out_ref):
    step = pl.program_id(0)

    m = m_ref[...]
    pm = pm_ref[...]

    m_sum = jnp.sum(m, axis=1, keepdims=True)
    pm_sum = jnp.sum(pm, axis=1, keepdims=True)
    c = (pm_sum * K).astype(jnp.int32)

    tv = jnp.sum(jnp.abs(m[:, 1:] - m[:, :-1]))

    bits = lax.bitcast_convert_type(m, jnp.int32)
    hi = (bits >> 16).astype(jnp.int16)

    def bodyH(i, h):
        cand = h | (jnp.int32(1) << (jnp.int32(13) - i))
        cnt = jnp.sum((hi >= cand.astype(jnp.int16)).astype(jnp.int16),
                      axis=1, keepdims=True)
        return jnp.where(cnt.astype(jnp.int32) >= c, cand, h)

    h = lax.fori_loop(0, 14, bodyH, jnp.zeros_like(c))
    h16 = h.astype(jnp.int16)
    cnt_hi_gt = jnp.sum((hi > h16).astype(jnp.int16), axis=1,
                        keepdims=True).astype(jnp.int32)
    lo = ((bits & jnp.int32(0xFFFF)) ^ jnp.int32(0x8000)).astype(jnp.int16)
    lo_m = jnp.where(hi == h16, lo, jnp.int16(-32768))
    rrem = c - cnt_hi_gt

    def bodyL(i, tu):
        cand_u = tu | (jnp.int32(1) << (jnp.int32(15) - i))
        cand_x = (cand_u ^ jnp.int32(0x8000)).astype(jnp.int16)
        cnt = jnp.sum((lo_m >= cand_x).astype(jnp.int16), axis=1, keepdims=True)
        return jnp.where(cnt.astype(jnp.int32) >= rrem, cand_u, tu)

    tu = lax.fori_loop(0, 16, bodyL, jnp.zeros_like(c))
    v = lax.bitcast_convert_type((h << 16) | tu, jnp.float32)
    gt = m > v
    cnt_gt = jnp.sum(gt.astype(jnp.int16), axis=1,
                     keepdims=True).astype(jnp.int32)
    sum_gt = jnp.sum(jnp.where(gt, m, 0.0), axis=1, keepdims=True)
    topc = sum_gt + (c - cnt_gt).astype(jnp.float32) * v
    row_spar = jnp.where(c == 0,
                         jnp.float32(L) - m_sum,
                         m_sum + c.astype(jnp.float32) - 2.0 * topc)
    partial = jnp.sum(row_spar) + tv / jnp.float32(B)

    @pl.when(step == 0)
    def _init():
        tgt = targets_ref[...]
        row0 = logits2_ref[0:1, :]
        row1 = logits2_ref[1:2, :]
        is0 = tgt == 0
        this = jnp.where(is0, row0, row1)
        other = jnp.maximum(jnp.where(is0, row1, row0), jnp.float32(NEG_BIG))
        nt = jnp.maximum(this - other + jnp.float32(NT_CONF), 0.0)
        comp = jnp.sum(nt) / jnp.float32(B)
        total = support_ref[0, 0] + comp + partial
        out_ref[...] = jnp.reshape(total, (1, 1))

    @pl.when(step != 0)
    def _acc():
        out_ref[...] = out_ref[...] + jnp.reshape(partial, (1, 1))


@functools.partial(jax.jit, static_argnames=())
def kernel(outputs_support, outputs_delete, targets, position_mask, masks):
    logits2 = outputs_delete[1].T
    tgt2 = targets.reshape(1, B)
    support = outputs_support.reshape(1, 2)
    out = pl.pallas_call(
        _loss_kernel,
        grid=(NUM_BLOCKS,),
        in_specs=[
            pl.BlockSpec((2, B), lambda i: (0, 0)),
            pl.BlockSpec((1, B), lambda i: (0, 0)),
            pl.BlockSpec((1, 2), lambda i: (0, 0)),
            pl.BlockSpec((ROW_BLOCK, L), lambda i: (i, 0)),
            pl.BlockSpec((ROW_BLOCK, L), lambda i: (i, 0)),
        ],
        out_specs=pl.BlockSpec((1, 1), lambda i: (0, 0)),
        out_shape=jax.ShapeDtypeStruct((1, 1), jnp.float32),
    )(logits2, tgt2, support, position_mask, masks)
    return out[0, 0]


# speedup vs baseline: 1.8471x; 1.3492x over previous
import functools

import jax
import jax.numpy as jnp
from jax import lax
from jax.experimental import pallas as pl
from jax.experimental.pallas import tpu as pltpu

B = 4096
L = 2048
K = 0.2
ROW_BLOCK = 512
NUM_BLOCKS = B // ROW_BLOCK
NT_CONF = 5.0
NEG_BIG = -12111.0


def _loss_kernel(logits2_ref, targets_ref, support_ref, pm_ref, m_ref, out_ref):
    step = pl.program_id(0)

    m = m_ref[...]
    pm = pm_ref[...]

    m_sum = jnp.sum(m, axis=1, keepdims=True)
    pm_sum = jnp.sum(pm, axis=1, keepdims=True)
    c = (pm_sum * K).astype(jnp.int32)

    tv = jnp.sum(jnp.abs(m[:, 1:] - m[:, :-1]))

    bits = lax.bitcast_convert_type(m, jnp.int32)
    ones_col = jnp.ones((L, 8), jnp.float32)
    c_f = c.astype(jnp.float32)

    def body(i, t):
        cand = t | (jnp.int32(1) << (jnp.int32(29) - i))
        ind = jnp.where(bits >= cand, 1.0, 0.0)
        cnt = jax.lax.dot_general(ind, ones_col, (((1,), (0,)), ((), ())),
                                  preferred_element_type=jnp.float32)[:, 0:1]
        return jnp.where(cnt >= c_f, cand, t)

    t = lax.fori_loop(0, 30, body, jnp.zeros_like(c))
    v = lax.bitcast_convert_type(t, jnp.float32)
    gt = bits > t
    cnt_gt = jnp.sum(gt.astype(jnp.int32), axis=1, keepdims=True)
    sum_gt = jnp.sum(jnp.where(gt, m, 0.0), axis=1, keepdims=True)
    topc = sum_gt + (c - cnt_gt).astype(jnp.float32) * v
    row_spar = jnp.where(c == 0,
                         jnp.float32(L) - m_sum,
                         m_sum + c.astype(jnp.float32) - 2.0 * topc)
    partial = jnp.sum(row_spar) + tv / jnp.float32(B)

    @pl.when(step == 0)
    def _init():
        tgt = targets_ref[...]
        row0 = logits2_ref[0:1, :]
        row1 = logits2_ref[1:2, :]
        is0 = tgt == 0
        this = jnp.where(is0, row0, row1)
        other = jnp.maximum(jnp.where(is0, row1, row0), jnp.float32(NEG_BIG))
        nt = jnp.maximum(this - other + jnp.float32(NT_CONF), 0.0)
        comp = jnp.sum(nt) / jnp.float32(B)
        total = support_ref[0, 0] + comp + partial
        out_ref[...] = jnp.reshape(total, (1, 1))

    @pl.when(step != 0)
    def _acc():
        out_ref[...] = out_ref[...] + jnp.reshape(partial, (1, 1))


@functools.partial(jax.jit, static_argnames=())
def kernel(outputs_support, outputs_delete, targets, position_mask, masks):
    logits2 = outputs_delete[1].T
    tgt2 = targets.reshape(1, B)
    support = outputs_support.reshape(1, 2)
    out = pl.pallas_call(
        _loss_kernel,
        grid=(NUM_BLOCKS,),
        in_specs=[
            pl.BlockSpec((2, B), lambda i: (0, 0)),
            pl.BlockSpec((1, B), lambda i: (0, 0)),
            pl.BlockSpec((1, 2), lambda i: (0, 0)),
            pl.BlockSpec((ROW_BLOCK, L), lambda i: (i, 0)),
            pl.BlockSpec((ROW_BLOCK, L), lambda i: (i, 0)),
        ],
        out_specs=pl.BlockSpec((1, 1), lambda i: (0, 0)),
        out_shape=jax.ShapeDtypeStruct((1, 1), jnp.float32),
    )(logits2, tgt2, support, position_mask, masks)
    return out[0, 0]


# speedup vs baseline: 2.6798x; 1.4508x over previous
import functools

import jax
import jax.numpy as jnp
from jax import lax
from jax.experimental import pallas as pl
from jax.experimental.pallas import tpu as pltpu

B = 4096
L = 2048
K = 0.2
ROW_BLOCK = 512
NUM_BLOCKS = B // ROW_BLOCK
NT_CONF = 5.0
NEG_BIG = -12111.0


def _loss_kernel(logits2_ref, targets_ref, support_ref, pm_ref, m_ref, out_ref):
    step = pl.program_id(0)

    m = m_ref[...]
    pm = pm_ref[...]

    m_sum = jnp.sum(m, axis=1, keepdims=True)
    pm_sum = jnp.sum(pm, axis=1, keepdims=True)
    c = (pm_sum * K).astype(jnp.int32)

    rolled = pltpu.roll(m, 1, 1)
    tv = (jnp.sum(jnp.abs(m - rolled))
          - jnp.sum(jnp.abs(m[:, 0:1] - m[:, L - 1:L])))

    def one(i, t):
        cand = t | (jnp.int32(1) << (jnp.int32(29) - i))
        candf = lax.bitcast_convert_type(cand, jnp.float32)
        cnt = jnp.sum((m >= candf).astype(jnp.int32), axis=1, keepdims=True)
        return jnp.where(cnt >= c, cand, t)

    def body(i, t):
        return one(2 * i + 1, one(2 * i, t))

    t = lax.fori_loop(0, 15, body, jnp.zeros_like(c))
    v = lax.bitcast_convert_type(t, jnp.float32)
    gt = m > v
    cnt_gt = jnp.sum(gt.astype(jnp.int32), axis=1, keepdims=True)
    sum_gt = jnp.sum(jnp.where(gt, m, 0.0), axis=1, keepdims=True)
    topc = sum_gt + (c - cnt_gt).astype(jnp.float32) * v
    row_spar = jnp.where(c == 0,
                         jnp.float32(L) - m_sum,
                         m_sum + c.astype(jnp.float32) - 2.0 * topc)
    partial = jnp.sum(row_spar) + tv / jnp.float32(B)

    @pl.when(step == 0)
    def _init():
        tgt = targets_ref[...]
        row0 = logits2_ref[0:1, :]
        row1 = logits2_ref[1:2, :]
        is0 = tgt == 0
        this = jnp.where(is0, row0, row1)
        other = jnp.maximum(jnp.where(is0, row1, row0), jnp.float32(NEG_BIG))
        nt = jnp.maximum(this - other + jnp.float32(NT_CONF), 0.0)
        comp = jnp.sum(nt) / jnp.float32(B)
        total = support_ref[0, 0] + comp + partial
        out_ref[...] = jnp.reshape(total, (1, 1))

    @pl.when(step != 0)
    def _acc():
        out_ref[...] = out_ref[...] + jnp.reshape(partial, (1, 1))


@functools.partial(jax.jit, static_argnames=())
def kernel(outputs_support, outputs_delete, targets, position_mask, masks):
    logits2 = outputs_delete[1].T
    tgt2 = targets.reshape(1, B)
    support = outputs_support.reshape(1, 2)
    out = pl.pallas_call(
        _loss_kernel,
        grid=(NUM_BLOCKS,),
        in_specs=[
            pl.BlockSpec((2, B), lambda i: (0, 0)),
            pl.BlockSpec((1, B), lambda i: (0, 0)),
            pl.BlockSpec((1, 2), lambda i: (0, 0)),
            pl.BlockSpec((ROW_BLOCK, L), lambda i: (i, 0)),
            pl.BlockSpec((ROW_BLOCK, L), lambda i: (i, 0)),
        ],
        out_specs=pl.BlockSpec((1, 1), lambda i: (0, 0)),
        out_shape=jax.ShapeDtypeStruct((1, 1), jnp.float32),
    )(logits2, tgt2, support, position_mask, masks)
    return out[0, 0]


# speedup vs baseline: 2.6817x; 1.0007x over previous
import functools

import jax
import jax.numpy as jnp
from jax import lax
from jax.experimental import pallas as pl
from jax.experimental.pallas import tpu as pltpu

B = 4096
L = 2048
K = 0.2
ROW_BLOCK = 1024
NUM_BLOCKS = B // ROW_BLOCK
NT_CONF = 5.0
NEG_BIG = -12111.0


def _loss_kernel(logits2_ref, targets_ref, support_ref, pm_ref, m_ref, out_ref):
    step = pl.program_id(0)

    m = m_ref[...]
    pm = pm_ref[...]

    m_sum = jnp.sum(m, axis=1, keepdims=True)
    pm_sum = jnp.sum(pm, axis=1, keepdims=True)
    c = (pm_sum * K).astype(jnp.int32)

    rolled = pltpu.roll(m, 1, 1)
    tv = (jnp.sum(jnp.abs(m - rolled))
          - jnp.sum(jnp.abs(m[:, 0:1] - m[:, L - 1:L])))

    def one(i, t):
        cand = t | (jnp.int32(1) << (jnp.int32(29) - i))
        candf = lax.bitcast_convert_type(cand, jnp.float32)
        cnt = jnp.sum((m >= candf).astype(jnp.int32), axis=1, keepdims=True)
        return jnp.where(cnt >= c, cand, t)

    def body(i, t):
        return one(2 * i + 1, one(2 * i, t))

    t = lax.fori_loop(0, 15, body, jnp.zeros_like(c))
    v = lax.bitcast_convert_type(t, jnp.float32)
    gt = m > v
    cnt_gt = jnp.sum(gt.astype(jnp.int32), axis=1, keepdims=True)
    sum_gt = jnp.sum(jnp.where(gt, m, 0.0), axis=1, keepdims=True)
    topc = sum_gt + (c - cnt_gt).astype(jnp.float32) * v
    row_spar = jnp.where(c == 0,
                         jnp.float32(L) - m_sum,
                         m_sum + c.astype(jnp.float32) - 2.0 * topc)
    partial = jnp.sum(row_spar) + tv / jnp.float32(B)

    @pl.when(step == 0)
    def _init():
        tgt = targets_ref[...]
        row0 = logits2_ref[0:1, :]
        row1 = logits2_ref[1:2, :]
        is0 = tgt == 0
        this = jnp.where(is0, row0, row1)
        other = jnp.maximum(jnp.where(is0, row1, row0), jnp.float32(NEG_BIG))
        nt = jnp.maximum(this - other + jnp.float32(NT_CONF), 0.0)
        comp = jnp.sum(nt) / jnp.float32(B)
        total = support_ref[0, 0] + comp + partial
        out_ref[...] = jnp.reshape(total, (1, 1))

    @pl.when(step != 0)
    def _acc():
        out_ref[...] = out_ref[...] + jnp.reshape(partial, (1, 1))


@functools.partial(jax.jit, static_argnames=())
def kernel(outputs_support, outputs_delete, targets, position_mask, masks):
    logits2 = outputs_delete[1].T
    tgt2 = targets.reshape(1, B)
    support = outputs_support.reshape(1, 2)
    out = pl.pallas_call(
        _loss_kernel,
        grid=(NUM_BLOCKS,),
        in_specs=[
            pl.BlockSpec((2, B), lambda i: (0, 0)),
            pl.BlockSpec((1, B), lambda i: (0, 0)),
            pl.BlockSpec((1, 2), lambda i: (0, 0)),
            pl.BlockSpec((ROW_BLOCK, L), lambda i: (i, 0)),
            pl.BlockSpec((ROW_BLOCK, L), lambda i: (i, 0)),
        ],
        out_specs=pl.BlockSpec((1, 1), lambda i: (0, 0)),
        out_shape=jax.ShapeDtypeStruct((1, 1), jnp.float32),
    )(logits2, tgt2, support, position_mask, masks)
    return out[0, 0]


# speedup vs baseline: 2.7573x; 1.0282x over previous
import functools

import jax
import jax.numpy as jnp
from jax import lax
from jax.experimental import pallas as pl
from jax.experimental.pallas import tpu as pltpu

B = 4096
L = 2048
K = 0.2
ROW_BLOCK = 1024
NUM_BLOCKS = B // ROW_BLOCK
NT_CONF = 5.0
NEG_BIG = -12111.0


def _loss_kernel(logits2_ref, targets_ref, support_ref, pm_ref, m_ref, out_ref):
    step = pl.program_id(0)

    m = m_ref[...]
    pm = pm_ref[...]

    m_sum = jnp.sum(m, axis=1, keepdims=True)
    pm_sum = jnp.sum(pm, axis=1, keepdims=True)
    c = (pm_sum * K).astype(jnp.int32)

    rolled = pltpu.roll(m, 1, 1)
    tv = (jnp.sum(jnp.abs(m - rolled))
          - jnp.sum(jnp.abs(m[:, 0:1] - m[:, L - 1:L])))

    def one(i, t):
        cand = t | (jnp.int32(1) << (jnp.int32(29) - i))
        candf = lax.bitcast_convert_type(cand, jnp.float32)
        cnt = jnp.sum((m >= candf).astype(jnp.int32), axis=1, keepdims=True)
        return jnp.where(cnt >= c, cand, t)

    def body(i, t):
        for u in range(5):
            t = one(5 * i + u, t)
        return t

    t = lax.fori_loop(0, 6, body, jnp.zeros_like(c))
    v = lax.bitcast_convert_type(t, jnp.float32)
    gt = m > v
    cnt_gt = jnp.sum(gt.astype(jnp.int32), axis=1, keepdims=True)
    sum_gt = jnp.sum(jnp.where(gt, m, 0.0), axis=1, keepdims=True)
    topc = sum_gt + (c - cnt_gt).astype(jnp.float32) * v
    row_spar = jnp.where(c == 0,
                         jnp.float32(L) - m_sum,
                         m_sum + c.astype(jnp.float32) - 2.0 * topc)
    partial = jnp.sum(row_spar) + tv / jnp.float32(B)

    @pl.when(step == 0)
    def _init():
        tgt = targets_ref[...]
        row0 = logits2_ref[0:1, :]
        row1 = logits2_ref[1:2, :]
        is0 = tgt == 0
        this = jnp.where(is0, row0, row1)
        other = jnp.maximum(jnp.where(is0, row1, row0), jnp.float32(NEG_BIG))
        nt = jnp.maximum(this - other + jnp.float32(NT_CONF), 0.0)
        comp = jnp.sum(nt) / jnp.float32(B)
        total = support_ref[0, 0] + comp + partial
        out_ref[...] = jnp.reshape(total, (1, 1))

    @pl.when(step != 0)
    def _acc():
        out_ref[...] = out_ref[...] + jnp.reshape(partial, (1, 1))


@functools.partial(jax.jit, static_argnames=())
def kernel(outputs_support, outputs_delete, targets, position_mask, masks):
    logits2 = outputs_delete[1].T
    tgt2 = targets.reshape(1, B)
    support = outputs_support.reshape(1, 2)
    out = pl.pallas_call(
        _loss_kernel,
        grid=(NUM_BLOCKS,),
        in_specs=[
            pl.BlockSpec((2, B), lambda i: (0, 0)),
            pl.BlockSpec((1, B), lambda i: (0, 0)),
            pl.BlockSpec((1, 2), lambda i: (0, 0)),
            pl.BlockSpec((ROW_BLOCK, L), lambda i: (i, 0)),
            pl.BlockSpec((ROW_BLOCK, L), lambda i: (i, 0)),
        ],
        out_specs=pl.BlockSpec((1, 1), lambda i: (0, 0)),
        out_shape=jax.ShapeDtypeStruct((1, 1), jnp.float32),
    )(logits2, tgt2, support, position_mask, masks)
    return out[0, 0]
